# baseline (speedup 1.0000x reference)
import jax
import jax.numpy as jnp
from jax.experimental import pallas as pl
from jax.experimental.pallas import tpu as pltpu

_L = 6
_D = 192
_N = 16
_R = 6


def _pool_proj_kernel(xv_ref, xr_ref, m_ref, wv_ref, wr_ref, out_ref, acc_ref):
    nb, ww, hh, cc = xv_ref.shape
    ck = pl.program_id(1)
    nk = pl.num_programs(1)
    dn = (((1,), (1,)), ((), ()))
    ysv = jnp.sum(xv_ref[...], axis=1)
    ysr = jnp.sum(xr_ref[...], axis=1)
    pm = m_ref[...]
    v = jax.lax.dot_general(pm, ysv, dn,
                            preferred_element_type=jnp.float32)
    r = jax.lax.dot_general(pm, ysr, dn,
                            preferred_element_type=jnp.float32)
    wv = wv_ref[pl.ds(ck * cc, cc), :]
    wr = wr_ref[pl.ds(ck * cc, cc), :]
    acc = (jnp.dot(v.reshape(_L * nb, cc), wv,
                   preferred_element_type=jnp.float32)
           + jnp.dot(r.reshape(_L * nb, cc), wr,
                     preferred_element_type=jnp.float32))

    @pl.when(ck == 0)
    def _():
        acc_ref[...] = acc

    @pl.when(ck > 0)
    def _():
        acc_ref[...] = acc_ref[...] + acc

    @pl.when(ck == nk - 1)
    def _():
        out_ref[...] = acc_ref[...].reshape(_L, nb, _D)


def _scan_tail_kernel(xt_ref, b1_ref, wdt_ref, dtwt_ref, wbc_ref, at_ref,
                      ds_ref, g_ref, bb_ref, wcat_ref, cb_ref, ov_ref, oi_ref):
    lq, bh, d = xt_ref.shape
    dnt = (((1,), (1,)), ((), ()))
    xt3 = xt_ref[...] + b1_ref[...]
    xt2 = xt3.reshape(lq * bh, d)
    dt_lin = jax.lax.dot_general(xt2, wdt_ref[...], dnt,
                                 preferred_element_type=jnp.float32)
    dts = jax.nn.softplus(jnp.dot(dt_lin, dtwt_ref[...],
                                  preferred_element_type=jnp.float32))
    bct = jax.lax.dot_general(wbc_ref[...], xt2, dnt,
                              preferred_element_type=jnp.float32)
    a_neg = -jnp.exp(at_ref[...])
    dts3 = dts.reshape(lq, bh, d)
    u = jnp.zeros((_N, bh, d), jnp.float32)
    ys = []
    for l in range(lq):
        dts_l = dts3[l]
        xt_l = xt3[l]
        da = jnp.exp(a_neg[:, None, :] * dts_l[None, :, :])
        bs_l = bct[:_N, l * bh:(l + 1) * bh]
        cs_l = bct[_N:, l * bh:(l + 1) * bh]
        u = da * u + (dts_l * xt_l)[None, :, :] * bs_l[:, :, None]
        ys.append(jnp.sum(u * cs_l[:, :, None], axis=0))
    y = jnp.stack(ys, axis=0) + xt3 * ds_ref[...]
    y2 = y.reshape(lq * bh, d)
    mu = jnp.mean(y2, axis=-1, keepdims=True)
    yc = y2 - mu
    var = jnp.mean(yc * yc, axis=-1, keepdims=True)
    yln = yc * jax.lax.rsqrt(var + 1e-5) * g_ref[...] + bb_ref[...]
    y3 = yln.reshape(lq, bh, d)
    accv = jnp.zeros((bh, cb_ref.shape[1]), jnp.float32) + cb_ref[...]
    acci = accv
    half = d // 2
    for l in range(lq):
        accv = accv + jnp.dot(y3[l][:, :half], wcat_ref[l],
                              preferred_element_type=jnp.float32)
        acci = acci + jnp.dot(y3[l][:, half:], wcat_ref[l],
                              preferred_element_type=jnp.float32)
    ov_ref[...] = jax.nn.sigmoid(accv)
    oi_ref[...] = jax.nn.sigmoid(acci)


def kernel(vis_feat, inf_feat, in_proj_w, in_proj_b, x_proj_w, dt_w,
           A_logs, Ds, ln_g, ln_b, cat_w, cat_b):
    B, C, H, W = vis_feat.shape
    hw = H * W
    seg = (H // _L) * W
    out_c = cat_w.shape[0]

    xv = jnp.transpose(vis_feat, (0, 3, 2, 1))
    xr = jnp.transpose(inf_feat, (0, 3, 2, 1))

    psel = jnp.repeat(jnp.eye(_L, dtype=jnp.float32), H // _L, axis=1)

    perm = jnp.concatenate([jnp.arange(0, _D, 2), jnp.arange(1, _D, 2)])
    w1 = in_proj_w[:_D][perm]
    wv4 = w1[:, 0::2].T / float(seg)
    wr4 = w1[:, 1::2].T / float(seg)
    b1 = in_proj_b[:_D][perm].reshape(1, _D)
    wdt = x_proj_w[:_R][:, perm]
    wbc = x_proj_w[_R:][:, perm]
    dt_wt = dt_w[perm].T
    a_t = A_logs[perm].T
    ds_p = Ds[perm].reshape(1, 1, _D)
    g_p = ln_g[perm].reshape(1, _D)
    lb_p = ln_b[perm].reshape(1, _D)
    wcat = jnp.transpose(cat_w.reshape(out_c, _D // 2, _L), (2, 1, 0))
    cb = cat_b.reshape(1, out_c)

    nb = 32
    cchunk = 256
    xt = pl.pallas_call(
        _pool_proj_kernel,
        grid=(B // nb, C // cchunk),
        in_specs=[
            pl.BlockSpec((nb, W, H, cchunk), lambda i, k: (i, 0, 0, k)),
            pl.BlockSpec((nb, W, H, cchunk), lambda i, k: (i, 0, 0, k)),
            pl.BlockSpec((_L, H), lambda i, k: (0, 0)),
            pl.BlockSpec((C, _D), lambda i, k: (0, 0)),
            pl.BlockSpec((C, _D), lambda i, k: (0, 0)),
        ],
        out_specs=pl.BlockSpec((_L, nb, _D), lambda i, k: (0, i, 0)),
        out_shape=jax.ShapeDtypeStruct((_L, B, _D), jnp.float32),
        scratch_shapes=[pltpu.VMEM((nb * _L, _D), jnp.float32)],
        compiler_params=pltpu.CompilerParams(
            dimension_semantics=("parallel", "arbitrary"),
            vmem_limit_bytes=60 * 1024 * 1024,
        ),
        name="pool_inproj",
    )(xv, xr, psel, wv4, wr4)

    bh = B
    ov, oi = pl.pallas_call(
        _scan_tail_kernel,
        grid=(1,),
        in_specs=[
            pl.BlockSpec((_L, bh, _D), lambda i: (0, i, 0)),
            pl.BlockSpec((1, _D), lambda i: (0, 0)),
            pl.BlockSpec((_R, _D), lambda i: (0, 0)),
            pl.BlockSpec((_R, _D), lambda i: (0, 0)),
            pl.BlockSpec((2 * _N, _D), lambda i: (0, 0)),
            pl.BlockSpec((_N, _D), lambda i: (0, 0)),
            pl.BlockSpec((1, 1, _D), lambda i: (0, 0, 0)),
            pl.BlockSpec((1, _D), lambda i: (0, 0)),
            pl.BlockSpec((1, _D), lambda i: (0, 0)),
            pl.BlockSpec((_L, _D // 2, out_c), lambda i: (0, 0, 0)),
            pl.BlockSpec((1, out_c), lambda i: (0, 0)),
        ],
        out_specs=[
            pl.BlockSpec((bh, out_c), lambda i: (i, 0)),
            pl.BlockSpec((bh, out_c), lambda i: (i, 0)),
        ],
        out_shape=[
            jax.ShapeDtypeStruct((B, out_c), jnp.float32),
            jax.ShapeDtypeStruct((B, out_c), jnp.float32),
        ],
        compiler_params=pltpu.CompilerParams(
            dimension_semantics=("parallel",),
            vmem_limit_bytes=64 * 1024 * 1024,
        ),
        name="scan_tail",
    )(xt, b1, wdt, dt_wt, wbc, a_t, ds_p, g_p, lb_p, wcat, cb)

    return ov[:, :, None, None], oi[:, :, None, None]


# speedup vs baseline: 1.0099x; 1.0099x over previous
import jax
import jax.numpy as jnp
from jax.experimental import pallas as pl
from jax.experimental.pallas import tpu as pltpu

_L = 6
_D = 192
_N = 16
_R = 6


def _pool_proj_kernel(xv_ref, xr_ref, m_ref, wv_ref, wr_ref, out_ref, acc_ref):
    nb, ww, hh, cc = xv_ref.shape
    ck = pl.program_id(1)
    nk = pl.num_programs(1)
    dn = (((1,), (1,)), ((), ()))
    ysv = jnp.sum(xv_ref[...], axis=1)
    ysr = jnp.sum(xr_ref[...], axis=1)
    pm = m_ref[...]
    v = jax.lax.dot_general(pm, ysv, dn,
                            preferred_element_type=jnp.float32)
    r = jax.lax.dot_general(pm, ysr, dn,
                            preferred_element_type=jnp.float32)
    wv = wv_ref[pl.ds(ck * cc, cc), :]
    wr = wr_ref[pl.ds(ck * cc, cc), :]
    acc = (jnp.dot(v.reshape(_L * nb, cc), wv,
                   preferred_element_type=jnp.float32)
           + jnp.dot(r.reshape(_L * nb, cc), wr,
                     preferred_element_type=jnp.float32))

    @pl.when(ck == 0)
    def _():
        acc_ref[...] = acc

    @pl.when(ck > 0)
    def _():
        acc_ref[...] = acc_ref[...] + acc

    @pl.when(ck == nk - 1)
    def _():
        out_ref[...] = acc_ref[...].reshape(_L, nb, _D)


def _scan_tail_kernel(xt_ref, b1_ref, wdt_ref, dtwt_ref, wbc_ref, at_ref,
                      ds_ref, g_ref, bb_ref, wcat_ref, cb_ref, ov_ref, oi_ref):
    lq, bh, d = xt_ref.shape
    dnt = (((1,), (1,)), ((), ()))
    xt3 = xt_ref[...] + b1_ref[...]
    xt2 = xt3.reshape(lq * bh, d)
    dt_lin = jax.lax.dot_general(xt2, wdt_ref[...], dnt,
                                 preferred_element_type=jnp.float32)
    dts = jax.nn.softplus(jnp.dot(dt_lin, dtwt_ref[...],
                                  preferred_element_type=jnp.float32))
    bct = jax.lax.dot_general(wbc_ref[...], xt2, dnt,
                              preferred_element_type=jnp.float32)
    a_neg = -jnp.exp(at_ref[...])
    dts3 = dts.reshape(lq, bh, d)
    u = None
    ys = []
    for l in range(lq):
        dts_l = dts3[l]
        xt_l = xt3[l]
        bs_l = bct[:_N, l * bh:(l + 1) * bh]
        cs_l = bct[_N:, l * bh:(l + 1) * bh]
        dbx = (dts_l * xt_l)[None, :, :] * bs_l[:, :, None]
        if u is None:
            u = dbx
        else:
            da = jnp.exp(a_neg[:, None, :] * dts_l[None, :, :])
            u = da * u + dbx
        ys.append(jnp.sum(u * cs_l[:, :, None], axis=0))
    y = jnp.stack(ys, axis=0) + xt3 * ds_ref[...]
    y2 = y.reshape(lq * bh, d)
    mu = jnp.mean(y2, axis=-1, keepdims=True)
    yc = y2 - mu
    var = jnp.mean(yc * yc, axis=-1, keepdims=True)
    yln = yc * jax.lax.rsqrt(var + 1e-5) * g_ref[...] + bb_ref[...]
    y3 = yln.reshape(lq, bh, d)
    accv = jnp.zeros((bh, cb_ref.shape[1]), jnp.float32) + cb_ref[...]
    acci = accv
    half = d // 2
    for l in range(lq):
        accv = accv + jnp.dot(y3[l][:, :half], wcat_ref[l],
                              preferred_element_type=jnp.float32)
        acci = acci + jnp.dot(y3[l][:, half:], wcat_ref[l],
                              preferred_element_type=jnp.float32)
    ov_ref[...] = jax.nn.sigmoid(accv)
    oi_ref[...] = jax.nn.sigmoid(acci)


def kernel(vis_feat, inf_feat, in_proj_w, in_proj_b, x_proj_w, dt_w,
           A_logs, Ds, ln_g, ln_b, cat_w, cat_b):
    B, C, H, W = vis_feat.shape
    hw = H * W
    seg = (H // _L) * W
    out_c = cat_w.shape[0]

    xv = jnp.transpose(vis_feat, (0, 3, 2, 1))
    xr = jnp.transpose(inf_feat, (0, 3, 2, 1))

    psel = jnp.repeat(jnp.eye(_L, dtype=jnp.float32), H // _L, axis=1)

    perm = jnp.concatenate([jnp.arange(0, _D, 2), jnp.arange(1, _D, 2)])
    w1 = in_proj_w[:_D][perm]
    wv4 = w1[:, 0::2].T / float(seg)
    wr4 = w1[:, 1::2].T / float(seg)
    b1 = in_proj_b[:_D][perm].reshape(1, _D)
    wdt = x_proj_w[:_R][:, perm]
    wbc = x_proj_w[_R:][:, perm]
    dt_wt = dt_w[perm].T
    a_t = A_logs[perm].T
    ds_p = Ds[perm].reshape(1, 1, _D)
    g_p = ln_g[perm].reshape(1, _D)
    lb_p = ln_b[perm].reshape(1, _D)
    wcat = jnp.transpose(cat_w.reshape(out_c, _D // 2, _L), (2, 1, 0))
    cb = cat_b.reshape(1, out_c)

    nb = 16
    cchunk = 512
    xt = pl.pallas_call(
        _pool_proj_kernel,
        grid=(B // nb, C // cchunk),
        in_specs=[
            pl.BlockSpec((nb, W, H, cchunk), lambda i, k: (i, 0, 0, k)),
            pl.BlockSpec((nb, W, H, cchunk), lambda i, k: (i, 0, 0, k)),
            pl.BlockSpec((_L, H), lambda i, k: (0, 0)),
            pl.BlockSpec((C, _D), lambda i, k: (0, 0)),
            pl.BlockSpec((C, _D), lambda i, k: (0, 0)),
        ],
        out_specs=pl.BlockSpec((_L, nb, _D), lambda i, k: (0, i, 0)),
        out_shape=jax.ShapeDtypeStruct((_L, B, _D), jnp.float32),
        scratch_shapes=[pltpu.VMEM((nb * _L, _D), jnp.float32)],
        compiler_params=pltpu.CompilerParams(
            dimension_semantics=("parallel", "arbitrary"),
            vmem_limit_bytes=60 * 1024 * 1024,
        ),
        name="pool_inproj",
    )(xv, xr, psel, wv4, wr4)

    bh = B
    ov, oi = pl.pallas_call(
        _scan_tail_kernel,
        grid=(1,),
        in_specs=[
            pl.BlockSpec((_L, bh, _D), lambda i: (0, i, 0)),
            pl.BlockSpec((1, _D), lambda i: (0, 0)),
            pl.BlockSpec((_R, _D), lambda i: (0, 0)),
            pl.BlockSpec((_R, _D), lambda i: (0, 0)),
            pl.BlockSpec((2 * _N, _D), lambda i: (0, 0)),
            pl.BlockSpec((_N, _D), lambda i: (0, 0)),
            pl.BlockSpec((1, 1, _D), lambda i: (0, 0, 0)),
            pl.BlockSpec((1, _D), lambda i: (0, 0)),
            pl.BlockSpec((1, _D), lambda i: (0, 0)),
            pl.BlockSpec((_L, _D // 2, out_c), lambda i: (0, 0, 0)),
            pl.BlockSpec((1, out_c), lambda i: (0, 0)),
        ],
        out_specs=[
            pl.BlockSpec((bh, out_c), lambda i: (i, 0)),
            pl.BlockSpec((bh, out_c), lambda i: (i, 0)),
        ],
        out_shape=[
            jax.ShapeDtypeStruct((B, out_c), jnp.float32),
            jax.ShapeDtypeStruct((B, out_c), jnp.float32),
        ],
        compiler_params=pltpu.CompilerParams(
            dimension_semantics=("parallel",),
            vmem_limit_bytes=64 * 1024 * 1024,
        ),
        name="scan_tail",
    )(xt, b1, wdt, dt_wt, wbc, a_t, ds_p, g_p, lb_p, wcat, cb)

    return ov[:, :, None, None], oi[:, :, None, None]
